# baseline (speedup 1.0000x reference)
import functools

import jax
import jax.numpy as jnp
import numpy as np
from jax import lax
from jax.experimental import pallas as pl
from jax.experimental.pallas import tpu as pltpu
from jax.experimental.pallas import tpu_sc as plsc

HIDDEN = 128
NG = 50
NGP = 64
CUTOFF = 10.0
MAXNB = 32
N = 10000
NPAD = 10240
GPAD = 104

R = 200
CB = 512
DB = 400

_OFFS = np.linspace(0.0, CUTOFF, NG).astype(np.float32)
_COEFF = float(-0.5 / (_OFFS[1] - _OFFS[0]) ** 2)
_OFFS_PAD = np.full((1, NGP), 1e9, np.float32)
_OFFS_PAD[0, :NG] = _OFFS


def _ssp(x):
    return jnp.maximum(x, 0.0) + jnp.log1p(jnp.exp(-jnp.abs(x))) - np.float32(np.log(2.0))


def _pack_bf16(x):
    lo = lax.bitcast_convert_type(x[:, :64].astype(jnp.bfloat16), jnp.uint16).astype(jnp.int32)
    hi = lax.bitcast_convert_type(x[:, 64:].astype(jnp.bfloat16), jnp.uint16).astype(jnp.int32)
    return lo | (hi << 16)


def _unpack_bf16(w):
    lo = lax.bitcast_convert_type(lax.shift_left(w, 16), jnp.float32)
    hi = lax.bitcast_convert_type(w & jnp.int32(-65536), jnp.float32)
    return lo, hi


def _sc_gather(table, idx, chunk):
    info = plsc.get_sparse_core_info()
    nc, ns = info.num_cores, info.num_subcores
    nw = nc * ns
    b = idx.shape[0]
    d = table.shape[1]
    bpw = b // nw
    nch = bpw // chunk
    assert bpw % chunk == 0 and b % nw == 0 and chunk % 8 == 0
    mesh = plsc.VectorSubcoreMesh(core_axis_name="c", subcore_axis_name="s")

    @functools.partial(
        pl.kernel,
        mesh=mesh,
        out_type=jax.ShapeDtypeStruct((b, d), table.dtype),
        scratch_types=[
            pltpu.VMEM((chunk,), jnp.int32),
            pltpu.VMEM((chunk,), jnp.int32),
            pltpu.VMEM((chunk, d), table.dtype),
            pltpu.VMEM((chunk, d), table.dtype),
            pltpu.SemaphoreType.DMA,
            pltpu.SemaphoreType.DMA,
            pltpu.SemaphoreType.DMA,
        ],
    )
    def k(table_hbm, idx_hbm, out_hbm, idx_v0, idx_v1, rows_v0, rows_v1,
          sem_g, sem_w0, sem_w1):
        wid = lax.axis_index("s") * nc + lax.axis_index("c")
        base = wid * bpw
        bufs = ((idx_v0, rows_v0, sem_w0), (idx_v1, rows_v1, sem_w1))

        def gather_in(c, bb):
            idx_v, rows_v, _ = bufs[bb]
            off = base + c * chunk
            pltpu.sync_copy(idx_hbm.at[pl.ds(off, chunk)], idx_v)
            pltpu.async_copy(table_hbm.at[idx_v], rows_v, sem_g).wait()

        def write_out(c, bb):
            _, rows_v, sem_w = bufs[bb]
            off = base + c * chunk
            pltpu.async_copy(rows_v, out_hbm.at[pl.ds(off, chunk)], sem_w)

        def drain(c, bb):
            _, rows_v, sem_w = bufs[bb]
            off = base + c * chunk
            pltpu.make_async_copy(rows_v, out_hbm.at[pl.ds(off, chunk)], sem_w).wait()

        if nch == 1:
            gather_in(0, 0)
            pltpu.sync_copy(rows_v0, out_hbm.at[pl.ds(base, chunk)])
            return

        assert nch % 2 == 0 and nch >= 4
        gather_in(0, 0)
        write_out(0, 0)
        gather_in(1, 1)
        write_out(1, 1)

        def body(c2, carry):
            c = c2 * 2
            for bb in range(2):
                drain(c + bb - 2, bb)
                gather_in(c + bb, bb)
                write_out(c + bb, bb)
            return carry

        lax.fori_loop(1, nch // 2, body, 0)
        drain(nch - 2, 0)
        drain(nch - 1, 1)

    return k(table, idx)


def _search_body(pos_r, batch_r, posT, batchT, src_o, d_o, cm_o):
    i = pl.program_id(0)
    rows0 = i * R
    px = pos_r[:, 0:1]
    py = pos_r[:, 1:2]
    pz = pos_r[:, 2:3]
    sqr = px * px + py * py + pz * pz
    brow = batch_r[...]
    g_first = batch_r[0, 0]
    g_last = batch_r[R - 1, 0]
    bcols_full = batchT[0:1, :]
    cidx_full = lax.broadcasted_iota(jnp.int32, (1, NPAD), 1)
    lo = jnp.min(jnp.where(bcols_full == g_first, cidx_full, NPAD))
    hi = jnp.max(jnp.where(bcols_full == g_last, cidx_full, -1)) + 1
    jlo = lo // CB
    jhi = (hi + CB - 1) // CB
    inf = jnp.float32(jnp.inf)
    best_v = jnp.full((R, MAXNB), inf, jnp.float32)
    best_i = jnp.zeros((R, MAXNB), jnp.int32)

    def body(j, carry):
        bv, bi = carry
        c0 = j * CB
        cx = posT[0:1, pl.ds(c0, CB)]
        cy = posT[1:2, pl.ds(c0, CB)]
        cz = posT[2:3, pl.ds(c0, CB)]
        bc = batchT[0:1, pl.ds(c0, CB)]
        sqc = cx * cx + cy * cy + cz * cz
        d2 = sqr + sqc - 2.0 * (px * cx + py * cy + pz * cz)
        col = lax.broadcasted_iota(jnp.int32, (R, CB), 1) + c0
        rowg = lax.broadcasted_iota(jnp.int32, (R, CB), 0) + rows0
        valid = (bc == brow) & (col != rowg)
        d2 = jnp.where(valid, d2, inf)
        cv = jnp.concatenate([bv, d2], axis=1)
        ci = jnp.concatenate([bi, col], axis=1)
        nv = []
        ni = []
        for _ in range(MAXNB):
            m = jnp.min(cv, axis=1, keepdims=True)
            sel = jnp.where(cv == m, ci, jnp.int32(2**30))
            mi = jnp.min(sel, axis=1, keepdims=True)
            nv.append(m)
            ni.append(mi)
            cv = jnp.where(ci == mi, inf, cv)
        return jnp.concatenate(nv, axis=1), jnp.concatenate(ni, axis=1)

    best_v, best_i = lax.fori_loop(jlo, jhi, body, (best_v, best_i))
    emask = best_v < jnp.float32(CUTOFF * CUTOFF)
    dsel = jnp.sqrt(jnp.where(emask, best_v, 1.0))
    cmv = jnp.where(emask, 0.5 * (jnp.cos(dsel * jnp.float32(np.pi / CUTOFF)) + 1.0), 0.0)
    src_o[...] = jnp.where(emask, best_i, 0)
    d_o[...] = dsel
    cm_o[...] = cmv


def _search(pos, batch):
    posT = jnp.zeros((8, NPAD), jnp.float32).at[0:3, :N].set(pos.T)
    batchT = jnp.full((8, NPAD), -1, jnp.int32).at[0:1, :N].set(batch[None, :])
    grid = N // R
    return pl.pallas_call(
        _search_body,
        grid=(grid,),
        in_specs=[
            pl.BlockSpec((R, 3), lambda i: (i, 0)),
            pl.BlockSpec((R, 1), lambda i: (i, 0)),
            pl.BlockSpec((8, NPAD), lambda i: (0, 0)),
            pl.BlockSpec((8, NPAD), lambda i: (0, 0)),
        ],
        out_specs=[
            pl.BlockSpec((R, MAXNB), lambda i: (i, 0)),
            pl.BlockSpec((R, MAXNB), lambda i: (i, 0)),
            pl.BlockSpec((R, MAXNB), lambda i: (i, 0)),
        ],
        out_shape=[
            jax.ShapeDtypeStruct((N, MAXNB), jnp.int32),
            jax.ShapeDtypeStruct((N, MAXNB), jnp.float32),
            jax.ShapeDtypeStruct((N, MAXNB), jnp.float32),
        ],
    )(pos, batch[:, None], posT, batchT)


def _matmul_body(x_r, w_r, o_r):
    o_r[...] = jnp.dot(x_r[...], w_r[...], preferred_element_type=jnp.float32)


def _matmul(x, w):
    rb = 1000
    grid = x.shape[0] // rb
    return pl.pallas_call(
        _matmul_body,
        grid=(grid,),
        in_specs=[
            pl.BlockSpec((rb, x.shape[1]), lambda i: (i, 0)),
            pl.BlockSpec(w.shape, lambda i: (0, 0)),
        ],
        out_specs=pl.BlockSpec((rb, w.shape[1]), lambda i: (i, 0)),
        out_shape=jax.ShapeDtypeStruct((x.shape[0], w.shape[1]), jnp.float32),
    )(x, w)


def _edge_body(d_r, cm_r, xs_r, h_r, offs_r, w1_r, b1_r, w2_r, b2_r,
               cw2_r, cb2_r, lw_r, lb_r, wn_r, h_o, x_o):
    coeff = jnp.float32(_COEFF)
    offs = offs_r[0:1, :]
    acc = jnp.zeros((DB, HIDDEN), jnp.float32)
    for k in range(MAXNB):
        dk = d_r[:, k:k + 1]
        ea = jnp.exp(coeff * (dk - offs) ** 2)
        t1 = _ssp(jnp.dot(ea, w1_r[...], preferred_element_type=jnp.float32) + b1_r[0:1, :])
        wk = jnp.dot(t1, w2_r[...], preferred_element_type=jnp.float32) + b2_r[0:1, :]
        wk = wk * cm_r[:, k:k + 1]
        acc = acc + xs_r[k] * wk
    x2 = _ssp(jnp.dot(acc, cw2_r[...], preferred_element_type=jnp.float32) + cb2_r[0:1, :])
    hn = h_r[...] + jnp.dot(x2, lw_r[...], preferred_element_type=jnp.float32) + lb_r[0:1, :]
    h_o[...] = hn
    x_o[...] = jnp.dot(hn, wn_r[...], preferred_element_type=jnp.float32)


def _edge_layer(d, cm, xs, h, offs, w1p, b1, w2, b2, cw2, cb2, lw, lb, wnext):
    grid = N // DB
    full = lambda a: pl.BlockSpec(a.shape, lambda i: (0,) * a.ndim)
    return pl.pallas_call(
        _edge_body,
        grid=(grid,),
        in_specs=[
            pl.BlockSpec((DB, MAXNB), lambda i: (i, 0)),
            pl.BlockSpec((DB, MAXNB), lambda i: (i, 0)),
            pl.BlockSpec((MAXNB, DB, HIDDEN), lambda i: (0, i, 0)),
            pl.BlockSpec((DB, HIDDEN), lambda i: (i, 0)),
            full(offs), full(w1p), full(b1), full(w2), full(b2),
            full(cw2), full(cb2), full(lw), full(lb), full(wnext),
        ],
        out_specs=[
            pl.BlockSpec((DB, HIDDEN), lambda i: (i, 0)),
            pl.BlockSpec((DB, HIDDEN), lambda i: (i, 0)),
        ],
        out_shape=[
            jax.ShapeDtypeStruct((N, HIDDEN), jnp.float32),
            jax.ShapeDtypeStruct((N, HIDDEN), jnp.float32),
        ],
    )(d, cm, xs, h, offs, w1p, b1, w2, b2, cw2, cb2, lw, lb, wnext)


def _readout_body(h_r, batch_r, o1_r, ob1_r, o2_r, ob2_r, out_r):
    i = pl.program_id(0)
    y = _ssp(jnp.dot(h_r[...], o1_r[...], preferred_element_type=jnp.float32) + ob1_r[0:1, :])
    y = jnp.dot(y, o2_r[...], preferred_element_type=jnp.float32) + ob2_r[0:1, :]
    giota = lax.broadcasted_iota(jnp.int32, (DB, GPAD), 1)
    oh = (batch_r[...] == giota).astype(jnp.float32)
    contrib = lax.dot_general(oh, y, (((0,), (0,)), ((), ())),
                              preferred_element_type=jnp.float32)

    @pl.when(i == 0)
    def _():
        out_r[...] = jnp.zeros_like(out_r)

    out_r[...] += contrib


def _readout(h, batch, o1, ob1, o2, ob2):
    grid = N // DB
    full = lambda a: pl.BlockSpec(a.shape, lambda i: (0,) * a.ndim)
    out = pl.pallas_call(
        _readout_body,
        grid=(grid,),
        in_specs=[
            pl.BlockSpec((DB, HIDDEN), lambda i: (i, 0)),
            pl.BlockSpec((DB, 1), lambda i: (i, 0)),
            full(o1), full(ob1), full(o2), full(ob2),
        ],
        out_specs=pl.BlockSpec((GPAD, HIDDEN), lambda i: (0, 0)),
        out_shape=jax.ShapeDtypeStruct((GPAD, HIDDEN), jnp.float32),
    )(h, batch[:, None], o1, ob1, o2, ob2)
    return out[:100]


def kernel(z, pos, batch, emb, mlp_w1, mlp_b1, mlp_w2, mlp_b2, conv_w1,
           conv_w2, conv_b2, lin_w, lin_b, out1_w, out1_b, out2_w, out2_b):
    T = mlp_w1.shape[0]
    batch = batch.astype(jnp.int32)
    z = z.astype(jnp.int32)

    z_pad = jnp.zeros((NPAD,), jnp.int32).at[:N].set(z)
    h = _sc_gather(emb, z_pad, 320)[:N]

    src, d, cm = _search(pos, batch)
    src_flat = src.T.reshape(-1)

    offs = jnp.asarray(_OFFS_PAD)
    w1p = jnp.zeros((T, NGP, HIDDEN), jnp.float32).at[:, :NG, :].set(mlp_w1)

    x = _matmul(h, conv_w1[0])
    for t in range(T):
        xs = _sc_gather(x, src_flat, 200).reshape(MAXNB, N, HIDDEN)
        wnext = conv_w1[(t + 1) % T]
        h, x = _edge_layer(
            d, cm, xs, h, offs,
            w1p[t], mlp_b1[t][None, :], mlp_w2[t], mlp_b2[t][None, :],
            conv_w2[t], conv_b2[t][None, :], lin_w[t], lin_b[t][None, :],
            wnext,
        )

    return _readout(h, batch, out1_w, out1_b[None, :], out2_w, out2_b[None, :])


# speedup vs baseline: 1.0700x; 1.0700x over previous
import functools

import jax
import jax.numpy as jnp
import numpy as np
from jax import lax
from jax.experimental import pallas as pl
from jax.experimental.pallas import tpu as pltpu
from jax.experimental.pallas import tpu_sc as plsc

HIDDEN = 128
NG = 50
NGP = 64
CUTOFF = 10.0
MAXNB = 32
N = 10000
NPAD = 10240
GPAD = 104

R = 200
CB = 512
DB = 400

_OFFS = np.linspace(0.0, CUTOFF, NG).astype(np.float32)
_COEFF = float(-0.5 / (_OFFS[1] - _OFFS[0]) ** 2)
_OFFS_PAD = np.full((1, NGP), 1e9, np.float32)
_OFFS_PAD[0, :NG] = _OFFS


def _sspn(x):
    return jnp.maximum(x, 0.0) + jnp.log1p(jnp.exp(-jnp.abs(x)))


def _sspf(x):
    return jnp.log1p(jnp.exp(x))


def _pack_bf16(x):
    lo = lax.bitcast_convert_type(x[:, :64].astype(jnp.bfloat16), jnp.uint16).astype(jnp.int32)
    hi = lax.bitcast_convert_type(x[:, 64:].astype(jnp.bfloat16), jnp.uint16).astype(jnp.int32)
    return lo | (hi << 16)


def _unpack_bf16(w):
    lo = lax.bitcast_convert_type(lax.shift_left(w, 16), jnp.float32)
    hi = lax.bitcast_convert_type(w & jnp.int32(-65536), jnp.float32)
    return lo, hi


def _sc_gather(table, idx, chunk):
    info = plsc.get_sparse_core_info()
    nc, ns = info.num_cores, info.num_subcores
    nw = nc * ns
    b = idx.shape[0]
    d = table.shape[1]
    bpw = b // nw
    nch = bpw // chunk
    assert bpw % chunk == 0 and b % nw == 0 and chunk % 8 == 0
    mesh = plsc.VectorSubcoreMesh(core_axis_name="c", subcore_axis_name="s")

    @functools.partial(
        pl.kernel,
        mesh=mesh,
        out_type=jax.ShapeDtypeStruct((b, d), table.dtype),
        scratch_types=[
            pltpu.VMEM((chunk,), jnp.int32),
            pltpu.VMEM((chunk,), jnp.int32),
            pltpu.VMEM((chunk, d), table.dtype),
            pltpu.VMEM((chunk, d), table.dtype),
            pltpu.SemaphoreType.DMA,
            pltpu.SemaphoreType.DMA,
            pltpu.SemaphoreType.DMA,
        ],
    )
    def k(table_hbm, idx_hbm, out_hbm, idx_v0, idx_v1, rows_v0, rows_v1,
          sem_g, sem_w0, sem_w1):
        wid = lax.axis_index("s") * nc + lax.axis_index("c")
        base = wid * bpw
        bufs = ((idx_v0, rows_v0, sem_w0), (idx_v1, rows_v1, sem_w1))

        def gather_in(c, bb):
            idx_v, rows_v, _ = bufs[bb]
            off = base + c * chunk
            pltpu.sync_copy(idx_hbm.at[pl.ds(off, chunk)], idx_v)
            pltpu.async_copy(table_hbm.at[idx_v], rows_v, sem_g).wait()

        def write_out(c, bb):
            _, rows_v, sem_w = bufs[bb]
            off = base + c * chunk
            pltpu.async_copy(rows_v, out_hbm.at[pl.ds(off, chunk)], sem_w)

        def drain(c, bb):
            _, rows_v, sem_w = bufs[bb]
            off = base + c * chunk
            pltpu.make_async_copy(rows_v, out_hbm.at[pl.ds(off, chunk)], sem_w).wait()

        if nch == 1:
            gather_in(0, 0)
            pltpu.sync_copy(rows_v0, out_hbm.at[pl.ds(base, chunk)])
            return

        assert nch % 2 == 0 and nch >= 4
        gather_in(0, 0)
        write_out(0, 0)
        gather_in(1, 1)
        write_out(1, 1)

        def body(c2, carry):
            c = c2 * 2
            for bb in range(2):
                drain(c + bb - 2, bb)
                gather_in(c + bb, bb)
                write_out(c + bb, bb)
            return carry

        lax.fori_loop(1, nch // 2, body, 0)
        drain(nch - 2, 0)
        drain(nch - 1, 1)

    return k(table, idx)


def _search_body(pos_r, batch_r, posT, batchT, src_o, d_o, cm_o):
    i = pl.program_id(0)
    rows0 = i * R
    px = pos_r[:, 0:1]
    py = pos_r[:, 1:2]
    pz = pos_r[:, 2:3]
    sqr = px * px + py * py + pz * pz
    brow = batch_r[...]
    g_first = batch_r[0, 0]
    g_last = batch_r[R - 1, 0]
    bcols_full = batchT[0:1, :]
    cidx_full = lax.broadcasted_iota(jnp.int32, (1, NPAD), 1)
    lo = jnp.min(jnp.where(bcols_full == g_first, cidx_full, NPAD))
    hi = jnp.max(jnp.where(bcols_full == g_last, cidx_full, -1)) + 1
    jlo = lo // CB
    jhi = (hi + CB - 1) // CB
    inf = jnp.float32(jnp.inf)
    best_v = jnp.full((R, MAXNB), inf, jnp.float32)
    best_i = jnp.zeros((R, MAXNB), jnp.int32)

    def body(j, carry):
        bv, bi = carry
        c0 = j * CB
        cx = posT[0:1, pl.ds(c0, CB)]
        cy = posT[1:2, pl.ds(c0, CB)]
        cz = posT[2:3, pl.ds(c0, CB)]
        bc = batchT[0:1, pl.ds(c0, CB)]
        sqc = cx * cx + cy * cy + cz * cz
        d2 = sqr + sqc - 2.0 * (px * cx + py * cy + pz * cz)
        col = lax.broadcasted_iota(jnp.int32, (R, CB), 1) + c0
        rowg = lax.broadcasted_iota(jnp.int32, (R, CB), 0) + rows0
        valid = (bc == brow) & (col != rowg)
        d2 = jnp.where(valid, d2, inf)
        cv = jnp.concatenate([bv, d2], axis=1)
        ci = jnp.concatenate([bi, col], axis=1)
        nv = []
        ni = []
        for _ in range(MAXNB):
            m = jnp.min(cv, axis=1, keepdims=True)
            sel = jnp.where(cv == m, ci, jnp.int32(2**30))
            mi = jnp.min(sel, axis=1, keepdims=True)
            nv.append(m)
            ni.append(mi)
            cv = jnp.where(ci == mi, inf, cv)
        return jnp.concatenate(nv, axis=1), jnp.concatenate(ni, axis=1)

    best_v, best_i = lax.fori_loop(jlo, jhi, body, (best_v, best_i))
    emask = best_v < jnp.float32(CUTOFF * CUTOFF)
    dsel = jnp.sqrt(jnp.where(emask, best_v, 1.0))
    cmv = jnp.where(emask, 0.5 * (jnp.cos(dsel * jnp.float32(np.pi / CUTOFF)) + 1.0), 0.0)
    src_o[...] = jnp.where(emask, best_i, 0)
    d_o[...] = dsel
    cm_o[...] = cmv


def _search(pos, batch):
    posT = jnp.zeros((8, NPAD), jnp.float32).at[0:3, :N].set(pos.T)
    batchT = jnp.full((8, NPAD), -1, jnp.int32).at[0:1, :N].set(batch[None, :])
    grid = N // R
    return pl.pallas_call(
        _search_body,
        grid=(grid,),
        in_specs=[
            pl.BlockSpec((R, 3), lambda i: (i, 0)),
            pl.BlockSpec((R, 1), lambda i: (i, 0)),
            pl.BlockSpec((8, NPAD), lambda i: (0, 0)),
            pl.BlockSpec((8, NPAD), lambda i: (0, 0)),
        ],
        out_specs=[
            pl.BlockSpec((R, MAXNB), lambda i: (i, 0)),
            pl.BlockSpec((R, MAXNB), lambda i: (i, 0)),
            pl.BlockSpec((R, MAXNB), lambda i: (i, 0)),
        ],
        out_shape=[
            jax.ShapeDtypeStruct((N, MAXNB), jnp.int32),
            jax.ShapeDtypeStruct((N, MAXNB), jnp.float32),
            jax.ShapeDtypeStruct((N, MAXNB), jnp.float32),
        ],
    )(pos, batch[:, None], posT, batchT)


def _matmul_body(x_r, w_r, o_r):
    o_r[...] = jnp.dot(x_r[...], w_r[...], preferred_element_type=jnp.float32)


def _matmul(x, w):
    rb = 1000
    grid = x.shape[0] // rb
    return pl.pallas_call(
        _matmul_body,
        grid=(grid,),
        in_specs=[
            pl.BlockSpec((rb, x.shape[1]), lambda i: (i, 0)),
            pl.BlockSpec(w.shape, lambda i: (0, 0)),
        ],
        out_specs=pl.BlockSpec((rb, w.shape[1]), lambda i: (i, 0)),
        out_shape=jax.ShapeDtypeStruct((x.shape[0], w.shape[1]), jnp.float32),
    )(x, w)


def _edge_body(d_r, cm_r, xs_r, h_r, offs_r, w1_r, b1_r, w2_r, b2_r,
               cw2_r, cb2_r, lw_r, lb_r, wn_r, h_o, x_o):
    coeff = jnp.float32(_COEFF)
    offs = offs_r[0:1, :]
    acc = jnp.zeros((DB, HIDDEN), jnp.float32)
    for k in range(MAXNB):
        dk = d_r[:, k:k + 1]
        ea = jnp.exp(coeff * (dk - offs) ** 2)
        t1 = _sspf(jnp.dot(ea, w1_r[...], preferred_element_type=jnp.float32) + b1_r[0:1, :])
        wk = jnp.dot(t1, w2_r[...], preferred_element_type=jnp.float32) + b2_r[0:1, :]
        wk = wk * cm_r[:, k:k + 1]
        acc = acc + xs_r[k] * wk
    x2 = _sspn(jnp.dot(acc, cw2_r[...], preferred_element_type=jnp.float32) + cb2_r[0:1, :])
    hn = h_r[...] + jnp.dot(x2, lw_r[...], preferred_element_type=jnp.float32) + lb_r[0:1, :]
    h_o[...] = hn
    x_o[...] = jnp.dot(hn, wn_r[...], preferred_element_type=jnp.float32)


def _edge_layer(d, cm, xs, h, offs, w1p, b1, w2, b2, cw2, cb2, lw, lb, wnext):
    grid = N // DB
    full = lambda a: pl.BlockSpec(a.shape, lambda i: (0,) * a.ndim)
    return pl.pallas_call(
        _edge_body,
        grid=(grid,),
        in_specs=[
            pl.BlockSpec((DB, MAXNB), lambda i: (i, 0)),
            pl.BlockSpec((DB, MAXNB), lambda i: (i, 0)),
            pl.BlockSpec((MAXNB, DB, HIDDEN), lambda i: (0, i, 0)),
            pl.BlockSpec((DB, HIDDEN), lambda i: (i, 0)),
            full(offs), full(w1p), full(b1), full(w2), full(b2),
            full(cw2), full(cb2), full(lw), full(lb), full(wnext),
        ],
        out_specs=[
            pl.BlockSpec((DB, HIDDEN), lambda i: (i, 0)),
            pl.BlockSpec((DB, HIDDEN), lambda i: (i, 0)),
        ],
        out_shape=[
            jax.ShapeDtypeStruct((N, HIDDEN), jnp.float32),
            jax.ShapeDtypeStruct((N, HIDDEN), jnp.float32),
        ],
    )(d, cm, xs, h, offs, w1p, b1, w2, b2, cw2, cb2, lw, lb, wnext)


def _readout_body(h_r, batch_r, o1_r, ob1_r, o2_r, ob2_r, out_r):
    i = pl.program_id(0)
    y = _sspn(jnp.dot(h_r[...], o1_r[...], preferred_element_type=jnp.float32) + ob1_r[0:1, :])
    y = jnp.dot(y, o2_r[...], preferred_element_type=jnp.float32) + ob2_r[0:1, :]
    giota = lax.broadcasted_iota(jnp.int32, (DB, GPAD), 1)
    oh = (batch_r[...] == giota).astype(jnp.float32)
    contrib = lax.dot_general(oh, y, (((0,), (0,)), ((), ())),
                              preferred_element_type=jnp.float32)

    @pl.when(i == 0)
    def _():
        out_r[...] = jnp.zeros_like(out_r)

    out_r[...] += contrib


def _readout(h, batch, o1, ob1, o2, ob2):
    grid = N // DB
    full = lambda a: pl.BlockSpec(a.shape, lambda i: (0,) * a.ndim)
    out = pl.pallas_call(
        _readout_body,
        grid=(grid,),
        in_specs=[
            pl.BlockSpec((DB, HIDDEN), lambda i: (i, 0)),
            pl.BlockSpec((DB, 1), lambda i: (i, 0)),
            full(o1), full(ob1), full(o2), full(ob2),
        ],
        out_specs=pl.BlockSpec((GPAD, HIDDEN), lambda i: (0, 0)),
        out_shape=jax.ShapeDtypeStruct((GPAD, HIDDEN), jnp.float32),
    )(h, batch[:, None], o1, ob1, o2, ob2)
    return out[:100]


def kernel(z, pos, batch, emb, mlp_w1, mlp_b1, mlp_w2, mlp_b2, conv_w1,
           conv_w2, conv_b2, lin_w, lin_b, out1_w, out1_b, out2_w, out2_b):
    T = mlp_w1.shape[0]
    batch = batch.astype(jnp.int32)
    z = z.astype(jnp.int32)

    z_pad = jnp.zeros((NPAD,), jnp.int32).at[:N].set(z)
    h = _sc_gather(emb, z_pad, 320)[:N]

    src, d, cm = _search(pos, batch)
    src_flat = src.T.reshape(-1)

    offs = jnp.asarray(_OFFS_PAD)
    w1p = jnp.zeros((T, NGP, HIDDEN), jnp.float32).at[:, :NG, :].set(mlp_w1)
    ln2 = np.float32(np.log(2.0))
    mlp_b2f = mlp_b2 - ln2 * mlp_w2.sum(axis=1)
    lin_bf = lin_b - ln2 * lin_w.sum(axis=1)
    out2_bf = out2_b - ln2 * out2_w.sum(axis=0)

    x = _matmul(h, conv_w1[0])
    for t in range(T):
        xs = _sc_gather(x, src_flat, 200).reshape(MAXNB, N, HIDDEN)
        wnext = conv_w1[(t + 1) % T]
        h, x = _edge_layer(
            d, cm, xs, h, offs,
            w1p[t], mlp_b1[t][None, :], mlp_w2[t], mlp_b2f[t][None, :],
            conv_w2[t], conv_b2[t][None, :], lin_w[t], lin_bf[t][None, :],
            wnext,
        )

    return _readout(h, batch, out1_w, out1_b[None, :], out2_w, out2_bf[None, :])


# speedup vs baseline: 1.0741x; 1.0038x over previous
import functools

import jax
import jax.numpy as jnp
import numpy as np
from jax import lax
from jax.experimental import pallas as pl
from jax.experimental.pallas import tpu as pltpu
from jax.experimental.pallas import tpu_sc as plsc

HIDDEN = 128
NG = 50
NGP = 64
CUTOFF = 10.0
MAXNB = 32
N = 10000
NPAD = 10240
GPAD = 104

R = 200
CB = 512
DB = 400

_OFFS = np.linspace(0.0, CUTOFF, NG).astype(np.float32)
_COEFF = float(-0.5 / (_OFFS[1] - _OFFS[0]) ** 2)
_OFFS_PAD = np.full((1, NGP), 1e9, np.float32)
_OFFS_PAD[0, :NG] = _OFFS


def _sspn(x):
    return jnp.maximum(x, 0.0) + jnp.log1p(jnp.exp(-jnp.abs(x)))


def _sspf(x):
    return jnp.log1p(jnp.exp(x))


def _pack_bf16(x):
    lo = lax.bitcast_convert_type(x[:, :64].astype(jnp.bfloat16), jnp.uint16).astype(jnp.int32)
    hi = lax.bitcast_convert_type(x[:, 64:].astype(jnp.bfloat16), jnp.uint16).astype(jnp.int32)
    return lo | (hi << 16)


def _unpack_bf16(w):
    lo = lax.bitcast_convert_type(lax.shift_left(w, 16), jnp.float32)
    hi = lax.bitcast_convert_type(w & jnp.int32(-65536), jnp.float32)
    return lo, hi


def _sc_gather(table, idx, chunk):
    info = plsc.get_sparse_core_info()
    nc, ns = info.num_cores, info.num_subcores
    nw = nc * ns
    b = idx.shape[0]
    d = table.shape[1]
    bpw = b // nw
    nch = bpw // chunk
    assert bpw % chunk == 0 and b % nw == 0 and chunk % 8 == 0
    mesh = plsc.VectorSubcoreMesh(core_axis_name="c", subcore_axis_name="s")

    @functools.partial(
        pl.kernel,
        mesh=mesh,
        out_type=jax.ShapeDtypeStruct((b, d), table.dtype),
        scratch_types=[
            pltpu.VMEM((chunk,), jnp.int32),
            pltpu.VMEM((chunk,), jnp.int32),
            pltpu.VMEM((chunk, d), table.dtype),
            pltpu.VMEM((chunk, d), table.dtype),
            pltpu.SemaphoreType.DMA,
            pltpu.SemaphoreType.DMA,
            pltpu.SemaphoreType.DMA,
        ],
    )
    def k(table_hbm, idx_hbm, out_hbm, idx_v0, idx_v1, rows_v0, rows_v1,
          sem_g, sem_w0, sem_w1):
        wid = lax.axis_index("s") * nc + lax.axis_index("c")
        base = wid * bpw
        bufs = ((idx_v0, rows_v0, sem_w0), (idx_v1, rows_v1, sem_w1))

        def gather_in(c, bb):
            idx_v, rows_v, _ = bufs[bb]
            off = base + c * chunk
            pltpu.sync_copy(idx_hbm.at[pl.ds(off, chunk)], idx_v)
            pltpu.async_copy(table_hbm.at[idx_v], rows_v, sem_g).wait()

        def write_out(c, bb):
            _, rows_v, sem_w = bufs[bb]
            off = base + c * chunk
            pltpu.async_copy(rows_v, out_hbm.at[pl.ds(off, chunk)], sem_w)

        def drain(c, bb):
            _, rows_v, sem_w = bufs[bb]
            off = base + c * chunk
            pltpu.make_async_copy(rows_v, out_hbm.at[pl.ds(off, chunk)], sem_w).wait()

        if nch == 1:
            gather_in(0, 0)
            pltpu.sync_copy(rows_v0, out_hbm.at[pl.ds(base, chunk)])
            return

        assert nch % 2 == 0 and nch >= 4
        gather_in(0, 0)
        write_out(0, 0)
        gather_in(1, 1)
        write_out(1, 1)

        def body(c2, carry):
            c = c2 * 2
            for bb in range(2):
                drain(c + bb - 2, bb)
                gather_in(c + bb, bb)
                write_out(c + bb, bb)
            return carry

        lax.fori_loop(1, nch // 2, body, 0)
        drain(nch - 2, 0)
        drain(nch - 1, 1)

    return k(table, idx)


def _search_body(pos_r, batch_r, posT, batchT, src_o, d_o, cm_o):
    i = pl.program_id(0)
    rows0 = i * R
    px = pos_r[:, 0:1]
    py = pos_r[:, 1:2]
    pz = pos_r[:, 2:3]
    sqr = px * px + py * py + pz * pz
    brow = batch_r[...]
    g_first = batch_r[0, 0]
    g_last = batch_r[R - 1, 0]
    bcols_full = batchT[0:1, :]
    cidx_full = lax.broadcasted_iota(jnp.int32, (1, NPAD), 1)
    lo = jnp.min(jnp.where(bcols_full == g_first, cidx_full, NPAD))
    hi = jnp.max(jnp.where(bcols_full == g_last, cidx_full, -1)) + 1
    jlo = lo // CB
    jhi = (hi + CB - 1) // CB
    inf = jnp.float32(jnp.inf)
    best_v = jnp.full((R, MAXNB), inf, jnp.float32)
    best_i = jnp.zeros((R, MAXNB), jnp.int32)

    def body(j, carry):
        bv, bi = carry
        c0 = j * CB
        cx = posT[0:1, pl.ds(c0, CB)]
        cy = posT[1:2, pl.ds(c0, CB)]
        cz = posT[2:3, pl.ds(c0, CB)]
        bc = batchT[0:1, pl.ds(c0, CB)]
        sqc = cx * cx + cy * cy + cz * cz
        cross = jnp.dot(pos_r[...], posT[0:3, pl.ds(c0, CB)],
                        preferred_element_type=jnp.float32)
        d2 = sqr + sqc - 2.0 * cross
        col = lax.broadcasted_iota(jnp.int32, (R, CB), 1) + c0
        rowg = lax.broadcasted_iota(jnp.int32, (R, CB), 0) + rows0
        valid = (bc == brow) & (col != rowg)
        d2 = jnp.where(valid, d2, inf)
        cv = jnp.concatenate([bv, d2], axis=1)
        ci = jnp.concatenate([bi, col], axis=1)
        nv = []
        ni = []
        for _ in range(MAXNB):
            m = jnp.min(cv, axis=1, keepdims=True)
            sel = jnp.where(cv == m, ci, jnp.int32(2**30))
            mi = jnp.min(sel, axis=1, keepdims=True)
            nv.append(m)
            ni.append(mi)
            cv = jnp.where(ci == mi, inf, cv)
        return jnp.concatenate(nv, axis=1), jnp.concatenate(ni, axis=1)

    best_v, best_i = lax.fori_loop(jlo, jhi, body, (best_v, best_i))
    emask = best_v < jnp.float32(CUTOFF * CUTOFF)
    dsel = jnp.sqrt(jnp.where(emask, best_v, 1.0))
    cmv = jnp.where(emask, 0.5 * (jnp.cos(dsel * jnp.float32(np.pi / CUTOFF)) + 1.0), 0.0)
    src_o[...] = jnp.where(emask, best_i, 0)
    d_o[...] = dsel
    cm_o[...] = cmv


def _search(pos, batch):
    posT = jnp.zeros((8, NPAD), jnp.float32).at[0:3, :N].set(pos.T)
    batchT = jnp.full((8, NPAD), -1, jnp.int32).at[0:1, :N].set(batch[None, :])
    grid = N // R
    return pl.pallas_call(
        _search_body,
        grid=(grid,),
        in_specs=[
            pl.BlockSpec((R, 3), lambda i: (i, 0)),
            pl.BlockSpec((R, 1), lambda i: (i, 0)),
            pl.BlockSpec((8, NPAD), lambda i: (0, 0)),
            pl.BlockSpec((8, NPAD), lambda i: (0, 0)),
        ],
        out_specs=[
            pl.BlockSpec((R, MAXNB), lambda i: (i, 0)),
            pl.BlockSpec((R, MAXNB), lambda i: (i, 0)),
            pl.BlockSpec((R, MAXNB), lambda i: (i, 0)),
        ],
        out_shape=[
            jax.ShapeDtypeStruct((N, MAXNB), jnp.int32),
            jax.ShapeDtypeStruct((N, MAXNB), jnp.float32),
            jax.ShapeDtypeStruct((N, MAXNB), jnp.float32),
        ],
    )(pos, batch[:, None], posT, batchT)


def _matmul_body(x_r, w_r, o_r):
    o_r[...] = jnp.dot(x_r[...], w_r[...], preferred_element_type=jnp.float32)


def _matmul(x, w):
    rb = 1000
    grid = x.shape[0] // rb
    return pl.pallas_call(
        _matmul_body,
        grid=(grid,),
        in_specs=[
            pl.BlockSpec((rb, x.shape[1]), lambda i: (i, 0)),
            pl.BlockSpec(w.shape, lambda i: (0, 0)),
        ],
        out_specs=pl.BlockSpec((rb, w.shape[1]), lambda i: (i, 0)),
        out_shape=jax.ShapeDtypeStruct((x.shape[0], w.shape[1]), jnp.float32),
    )(x, w)


def _edge_body(d_r, cm_r, xs_r, h_r, offs_r, w1_r, b1_r, w2_r, b2_r,
               cw2_r, cb2_r, lw_r, lb_r, wn_r, h_o, x_o):
    coeff = jnp.float32(_COEFF)
    offs = offs_r[0:1, :]
    acc = jnp.zeros((DB, HIDDEN), jnp.float32)
    for k in range(MAXNB):
        dk = d_r[:, k:k + 1]
        ea = jnp.exp(coeff * (dk - offs) ** 2)
        t1 = _sspf(jnp.dot(ea, w1_r[...], preferred_element_type=jnp.float32) + b1_r[0:1, :])
        wk = jnp.dot(t1, w2_r[...], preferred_element_type=jnp.float32) + b2_r[0:1, :]
        wk = wk * cm_r[:, k:k + 1]
        acc = acc + xs_r[k] * wk
    x2 = _sspn(jnp.dot(acc, cw2_r[...], preferred_element_type=jnp.float32) + cb2_r[0:1, :])
    hn = h_r[...] + jnp.dot(x2, lw_r[...], preferred_element_type=jnp.float32) + lb_r[0:1, :]
    h_o[...] = hn
    x_o[...] = jnp.dot(hn, wn_r[...], preferred_element_type=jnp.float32)


def _edge_layer(d, cm, xs, h, offs, w1p, b1, w2, b2, cw2, cb2, lw, lb, wnext):
    grid = N // DB
    full = lambda a: pl.BlockSpec(a.shape, lambda i: (0,) * a.ndim)
    return pl.pallas_call(
        _edge_body,
        grid=(grid,),
        in_specs=[
            pl.BlockSpec((DB, MAXNB), lambda i: (i, 0)),
            pl.BlockSpec((DB, MAXNB), lambda i: (i, 0)),
            pl.BlockSpec((MAXNB, DB, HIDDEN), lambda i: (0, i, 0)),
            pl.BlockSpec((DB, HIDDEN), lambda i: (i, 0)),
            full(offs), full(w1p), full(b1), full(w2), full(b2),
            full(cw2), full(cb2), full(lw), full(lb), full(wnext),
        ],
        out_specs=[
            pl.BlockSpec((DB, HIDDEN), lambda i: (i, 0)),
            pl.BlockSpec((DB, HIDDEN), lambda i: (i, 0)),
        ],
        out_shape=[
            jax.ShapeDtypeStruct((N, HIDDEN), jnp.float32),
            jax.ShapeDtypeStruct((N, HIDDEN), jnp.float32),
        ],
    )(d, cm, xs, h, offs, w1p, b1, w2, b2, cw2, cb2, lw, lb, wnext)


def _readout_body(h_r, batch_r, o1_r, ob1_r, o2_r, ob2_r, out_r):
    i = pl.program_id(0)
    y = _sspn(jnp.dot(h_r[...], o1_r[...], preferred_element_type=jnp.float32) + ob1_r[0:1, :])
    y = jnp.dot(y, o2_r[...], preferred_element_type=jnp.float32) + ob2_r[0:1, :]
    giota = lax.broadcasted_iota(jnp.int32, (DB, GPAD), 1)
    oh = (batch_r[...] == giota).astype(jnp.float32)
    contrib = lax.dot_general(oh, y, (((0,), (0,)), ((), ())),
                              preferred_element_type=jnp.float32)

    @pl.when(i == 0)
    def _():
        out_r[...] = jnp.zeros_like(out_r)

    out_r[...] += contrib


def _readout(h, batch, o1, ob1, o2, ob2):
    grid = N // DB
    full = lambda a: pl.BlockSpec(a.shape, lambda i: (0,) * a.ndim)
    out = pl.pallas_call(
        _readout_body,
        grid=(grid,),
        in_specs=[
            pl.BlockSpec((DB, HIDDEN), lambda i: (i, 0)),
            pl.BlockSpec((DB, 1), lambda i: (i, 0)),
            full(o1), full(ob1), full(o2), full(ob2),
        ],
        out_specs=pl.BlockSpec((GPAD, HIDDEN), lambda i: (0, 0)),
        out_shape=jax.ShapeDtypeStruct((GPAD, HIDDEN), jnp.float32),
    )(h, batch[:, None], o1, ob1, o2, ob2)
    return out[:100]


def kernel(z, pos, batch, emb, mlp_w1, mlp_b1, mlp_w2, mlp_b2, conv_w1,
           conv_w2, conv_b2, lin_w, lin_b, out1_w, out1_b, out2_w, out2_b):
    T = mlp_w1.shape[0]
    batch = batch.astype(jnp.int32)
    z = z.astype(jnp.int32)

    z_pad = jnp.zeros((NPAD,), jnp.int32).at[:N].set(z)
    h = _sc_gather(emb, z_pad, 320)[:N]

    src, d, cm = _search(pos, batch)
    src_flat = src.T.reshape(-1)

    offs = jnp.asarray(_OFFS_PAD)
    w1p = jnp.zeros((T, NGP, HIDDEN), jnp.float32).at[:, :NG, :].set(mlp_w1)
    ln2 = np.float32(np.log(2.0))
    mlp_b2f = mlp_b2 - ln2 * mlp_w2.sum(axis=1)
    lin_bf = lin_b - ln2 * lin_w.sum(axis=1)
    out2_bf = out2_b - ln2 * out2_w.sum(axis=0)

    x = _matmul(h, conv_w1[0])
    for t in range(T):
        xs = _sc_gather(x, src_flat, 200).reshape(MAXNB, N, HIDDEN)
        wnext = conv_w1[(t + 1) % T]
        h, x = _edge_layer(
            d, cm, xs, h, offs,
            w1p[t], mlp_b1[t][None, :], mlp_w2[t], mlp_b2f[t][None, :],
            conv_w2[t], conv_b2[t][None, :], lin_w[t], lin_bf[t][None, :],
            wnext,
        )

    return _readout(h, batch, out1_w, out1_b[None, :], out2_w, out2_bf[None, :])


# speedup vs baseline: 1.0742x; 1.0001x over previous
import functools

import jax
import jax.numpy as jnp
import numpy as np
from jax import lax
from jax.experimental import pallas as pl
from jax.experimental.pallas import tpu as pltpu
from jax.experimental.pallas import tpu_sc as plsc

HIDDEN = 128
NG = 50
NGP = 64
CUTOFF = 10.0
MAXNB = 32
N = 10000
NPAD = 10240
GPAD = 104

R = 200
CB = 512
DB = 1000

_OFFS = np.linspace(0.0, CUTOFF, NG).astype(np.float32)
_COEFF = float(-0.5 / (_OFFS[1] - _OFFS[0]) ** 2)
_OFFS_PAD = np.full((1, NGP), 1e9, np.float32)
_OFFS_PAD[0, :NG] = _OFFS


def _sspn(x):
    return jnp.maximum(x, 0.0) + jnp.log1p(jnp.exp(-jnp.abs(x)))


def _sspf(x):
    return jnp.log1p(jnp.exp(x))


def _sc_gather(table, idx, chunk):
    info = plsc.get_sparse_core_info()
    nc, ns = info.num_cores, info.num_subcores
    nw = nc * ns
    b = idx.shape[0]
    d = table.shape[1]
    bpw = b // nw
    nch = bpw // chunk
    assert bpw % chunk == 0 and b % nw == 0 and chunk % 8 == 0
    mesh = plsc.VectorSubcoreMesh(core_axis_name="c", subcore_axis_name="s")

    @functools.partial(
        pl.kernel,
        mesh=mesh,
        out_type=jax.ShapeDtypeStruct((b, d), table.dtype),
        scratch_types=[
            pltpu.VMEM((chunk,), jnp.int32),
            pltpu.VMEM((chunk,), jnp.int32),
            pltpu.VMEM((chunk, d), table.dtype),
            pltpu.VMEM((chunk, d), table.dtype),
            pltpu.SemaphoreType.DMA,
            pltpu.SemaphoreType.DMA,
            pltpu.SemaphoreType.DMA,
        ],
    )
    def k(table_hbm, idx_hbm, out_hbm, idx_v0, idx_v1, rows_v0, rows_v1,
          sem_g, sem_w0, sem_w1):
        wid = lax.axis_index("s") * nc + lax.axis_index("c")
        base = wid * bpw
        bufs = ((idx_v0, rows_v0, sem_w0), (idx_v1, rows_v1, sem_w1))

        def gather_in(c, bb):
            idx_v, rows_v, _ = bufs[bb]
            off = base + c * chunk
            pltpu.sync_copy(idx_hbm.at[pl.ds(off, chunk)], idx_v)
            pltpu.async_copy(table_hbm.at[idx_v], rows_v, sem_g).wait()

        def write_out(c, bb):
            _, rows_v, sem_w = bufs[bb]
            off = base + c * chunk
            pltpu.async_copy(rows_v, out_hbm.at[pl.ds(off, chunk)], sem_w)

        def drain(c, bb):
            _, rows_v, sem_w = bufs[bb]
            off = base + c * chunk
            pltpu.make_async_copy(rows_v, out_hbm.at[pl.ds(off, chunk)], sem_w).wait()

        if nch == 1:
            gather_in(0, 0)
            pltpu.sync_copy(rows_v0, out_hbm.at[pl.ds(base, chunk)])
            return

        assert nch % 2 == 0 and nch >= 4
        gather_in(0, 0)
        write_out(0, 0)
        gather_in(1, 1)
        write_out(1, 1)

        def body(c2, carry):
            c = c2 * 2
            for bb in range(2):
                drain(c + bb - 2, bb)
                gather_in(c + bb, bb)
                write_out(c + bb, bb)
            return carry

        lax.fori_loop(1, nch // 2, body, 0)
        drain(nch - 2, 0)
        drain(nch - 1, 1)

    return k(table, idx)


def _search_body(pos_r, batch_r, posT, batchT, src_o, d_o, cm_o):
    i = pl.program_id(0)
    rows0 = i * R
    px = pos_r[:, 0:1]
    py = pos_r[:, 1:2]
    pz = pos_r[:, 2:3]
    sqr = px * px + py * py + pz * pz
    brow = batch_r[...]
    g_first = batch_r[0, 0]
    g_last = batch_r[R - 1, 0]
    bcols_full = batchT[0:1, :]
    cidx_full = lax.broadcasted_iota(jnp.int32, (1, NPAD), 1)
    lo = jnp.min(jnp.where(bcols_full == g_first, cidx_full, NPAD))
    hi = jnp.max(jnp.where(bcols_full == g_last, cidx_full, -1)) + 1
    jlo = lo // CB
    jhi = (hi + CB - 1) // CB
    inf = jnp.float32(jnp.inf)
    best_v = jnp.full((R, MAXNB), inf, jnp.float32)
    best_i = jnp.zeros((R, MAXNB), jnp.int32)

    def body(j, carry):
        bv, bi = carry
        c0 = j * CB
        cx = posT[0:1, pl.ds(c0, CB)]
        cy = posT[1:2, pl.ds(c0, CB)]
        cz = posT[2:3, pl.ds(c0, CB)]
        bc = batchT[0:1, pl.ds(c0, CB)]
        sqc = cx * cx + cy * cy + cz * cz
        cross = jnp.dot(pos_r[...], posT[0:3, pl.ds(c0, CB)],
                        preferred_element_type=jnp.float32)
        d2 = sqr + sqc - 2.0 * cross
        col = lax.broadcasted_iota(jnp.int32, (R, CB), 1) + c0
        rowg = lax.broadcasted_iota(jnp.int32, (R, CB), 0) + rows0
        valid = (bc == brow) & (col != rowg)
        d2 = jnp.where(valid, d2, inf)
        cv = jnp.concatenate([bv, d2], axis=1)
        ci = jnp.concatenate([bi, col], axis=1)
        nv = []
        ni = []
        for _ in range(MAXNB):
            m = jnp.min(cv, axis=1, keepdims=True)
            sel = jnp.where(cv == m, ci, jnp.int32(2**30))
            mi = jnp.min(sel, axis=1, keepdims=True)
            nv.append(m)
            ni.append(mi)
            cv = jnp.where(ci == mi, inf, cv)
        return jnp.concatenate(nv, axis=1), jnp.concatenate(ni, axis=1)

    best_v, best_i = lax.fori_loop(jlo, jhi, body, (best_v, best_i))
    emask = best_v < jnp.float32(CUTOFF * CUTOFF)
    dsel = jnp.sqrt(jnp.where(emask, best_v, 1.0))
    cmv = jnp.where(emask, 0.5 * (jnp.cos(dsel * jnp.float32(np.pi / CUTOFF)) + 1.0), 0.0)
    src_o[...] = jnp.where(emask, best_i, 0)
    d_o[...] = dsel
    cm_o[...] = cmv


def _search(pos, batch):
    posT = jnp.zeros((8, NPAD), jnp.float32).at[0:3, :N].set(pos.T)
    batchT = jnp.full((8, NPAD), -1, jnp.int32).at[0:1, :N].set(batch[None, :])
    grid = N // R
    return pl.pallas_call(
        _search_body,
        grid=(grid,),
        in_specs=[
            pl.BlockSpec((R, 3), lambda i: (i, 0)),
            pl.BlockSpec((R, 1), lambda i: (i, 0)),
            pl.BlockSpec((8, NPAD), lambda i: (0, 0)),
            pl.BlockSpec((8, NPAD), lambda i: (0, 0)),
        ],
        out_specs=[
            pl.BlockSpec((R, MAXNB), lambda i: (i, 0)),
            pl.BlockSpec((R, MAXNB), lambda i: (i, 0)),
            pl.BlockSpec((R, MAXNB), lambda i: (i, 0)),
        ],
        out_shape=[
            jax.ShapeDtypeStruct((N, MAXNB), jnp.int32),
            jax.ShapeDtypeStruct((N, MAXNB), jnp.float32),
            jax.ShapeDtypeStruct((N, MAXNB), jnp.float32),
        ],
    )(pos, batch[:, None], posT, batchT)


def _matmul_body(x_r, w_r, o_r):
    o_r[...] = jnp.dot(x_r[...], w_r[...], preferred_element_type=jnp.float32)


def _matmul(x, w):
    rb = 1000
    grid = x.shape[0] // rb
    return pl.pallas_call(
        _matmul_body,
        grid=(grid,),
        in_specs=[
            pl.BlockSpec((rb, x.shape[1]), lambda i: (i, 0)),
            pl.BlockSpec(w.shape, lambda i: (0, 0)),
        ],
        out_specs=pl.BlockSpec((rb, w.shape[1]), lambda i: (i, 0)),
        out_shape=jax.ShapeDtypeStruct((x.shape[0], w.shape[1]), jnp.float32),
    )(x, w)


def _edge_body(d_r, cm_r, xs_r, h_r, offs_r, w1_r, b1_r, w2_r, b2_r,
               cw2_r, cb2_r, lw_r, lb_r, wn_r, h_o, x_o):
    coeff = jnp.float32(_COEFF)
    offs = offs_r[0:1, :]
    acc = jnp.zeros((DB, HIDDEN), jnp.float32)
    for k in range(MAXNB):
        dk = d_r[:, k:k + 1]
        ea = jnp.exp(coeff * (dk - offs) ** 2)
        t1 = _sspf(jnp.dot(ea, w1_r[...], preferred_element_type=jnp.float32) + b1_r[0:1, :])
        wk = jnp.dot(t1, w2_r[...], preferred_element_type=jnp.float32) + b2_r[0:1, :]
        wk = wk * cm_r[:, k:k + 1]
        acc = acc + xs_r[k] * wk
    x2 = _sspn(jnp.dot(acc, cw2_r[...], preferred_element_type=jnp.float32) + cb2_r[0:1, :])
    hn = h_r[...] + jnp.dot(x2, lw_r[...], preferred_element_type=jnp.float32) + lb_r[0:1, :]
    h_o[...] = hn
    x_o[...] = jnp.dot(hn, wn_r[...], preferred_element_type=jnp.float32)


def _edge_layer(d, cm, xs, h, offs, w1p, b1, w2, b2, cw2, cb2, lw, lb, wnext):
    grid = N // DB
    full = lambda a: pl.BlockSpec(a.shape, lambda i: (0,) * a.ndim)
    return pl.pallas_call(
        _edge_body,
        grid=(grid,),
        in_specs=[
            pl.BlockSpec((DB, MAXNB), lambda i: (i, 0)),
            pl.BlockSpec((DB, MAXNB), lambda i: (i, 0)),
            pl.BlockSpec((MAXNB, DB, HIDDEN), lambda i: (0, i, 0)),
            pl.BlockSpec((DB, HIDDEN), lambda i: (i, 0)),
            full(offs), full(w1p), full(b1), full(w2), full(b2),
            full(cw2), full(cb2), full(lw), full(lb), full(wnext),
        ],
        out_specs=[
            pl.BlockSpec((DB, HIDDEN), lambda i: (i, 0)),
            pl.BlockSpec((DB, HIDDEN), lambda i: (i, 0)),
        ],
        out_shape=[
            jax.ShapeDtypeStruct((N, HIDDEN), jnp.float32),
            jax.ShapeDtypeStruct((N, HIDDEN), jnp.float32),
        ],
    )(d, cm, xs, h, offs, w1p, b1, w2, b2, cw2, cb2, lw, lb, wnext)


def _readout_body(h_r, batch_r, o1_r, ob1_r, o2_r, ob2_r, out_r):
    i = pl.program_id(0)
    y = _sspn(jnp.dot(h_r[...], o1_r[...], preferred_element_type=jnp.float32) + ob1_r[0:1, :])
    y = jnp.dot(y, o2_r[...], preferred_element_type=jnp.float32) + ob2_r[0:1, :]
    giota = lax.broadcasted_iota(jnp.int32, (DB, GPAD), 1)
    oh = (batch_r[...] == giota).astype(jnp.float32)
    contrib = lax.dot_general(oh, y, (((0,), (0,)), ((), ())),
                              preferred_element_type=jnp.float32)

    @pl.when(i == 0)
    def _():
        out_r[...] = jnp.zeros_like(out_r)

    out_r[...] += contrib


def _readout(h, batch, o1, ob1, o2, ob2):
    grid = N // DB
    full = lambda a: pl.BlockSpec(a.shape, lambda i: (0,) * a.ndim)
    out = pl.pallas_call(
        _readout_body,
        grid=(grid,),
        in_specs=[
            pl.BlockSpec((DB, HIDDEN), lambda i: (i, 0)),
            pl.BlockSpec((DB, 1), lambda i: (i, 0)),
            full(o1), full(ob1), full(o2), full(ob2),
        ],
        out_specs=pl.BlockSpec((GPAD, HIDDEN), lambda i: (0, 0)),
        out_shape=jax.ShapeDtypeStruct((GPAD, HIDDEN), jnp.float32),
    )(h, batch[:, None], o1, ob1, o2, ob2)
    return out[:100]


def kernel(z, pos, batch, emb, mlp_w1, mlp_b1, mlp_w2, mlp_b2, conv_w1,
           conv_w2, conv_b2, lin_w, lin_b, out1_w, out1_b, out2_w, out2_b):
    T = mlp_w1.shape[0]
    batch = batch.astype(jnp.int32)
    z = z.astype(jnp.int32)

    z_pad = jnp.zeros((NPAD,), jnp.int32).at[:N].set(z)
    h = _sc_gather(emb, z_pad, 320)[:N]

    src, d, cm = _search(pos, batch)
    src_flat = src.T.reshape(-1)

    offs = jnp.asarray(_OFFS_PAD)
    w1p = jnp.zeros((T, NGP, HIDDEN), jnp.float32).at[:, :NG, :].set(mlp_w1)
    ln2 = np.float32(np.log(2.0))
    mlp_b2f = mlp_b2 - ln2 * mlp_w2.sum(axis=1)
    lin_bf = lin_b - ln2 * lin_w.sum(axis=1)
    out2_bf = out2_b - ln2 * out2_w.sum(axis=0)

    x = _matmul(h, conv_w1[0])
    for t in range(T):
        xs = _sc_gather(x, src_flat, 200).reshape(MAXNB, N, HIDDEN)
        wnext = conv_w1[(t + 1) % T]
        h, x = _edge_layer(
            d, cm, xs, h, offs,
            w1p[t], mlp_b1[t][None, :], mlp_w2[t], mlp_b2f[t][None, :],
            conv_w2[t], conv_b2[t][None, :], lin_w[t], lin_bf[t][None, :],
            wnext,
        )

    return _readout(h, batch, out1_w, out1_b[None, :], out2_w, out2_bf[None, :])
